# Initial kernel scaffold; baseline (speedup 1.0000x reference)
import jax
import jax.numpy as jnp
from jax.experimental import pallas as pl


def kernel(indices, table):
    raise NotImplementedError("write your pallas kernel here")


# speedup vs baseline: 1.0088x; 1.0088x over previous
import jax
import jax.numpy as jnp
from jax import lax
from jax.experimental import pallas as pl
from jax.experimental.pallas import tpu as pltpu
from jax.experimental.pallas import tpu_sc as plsc

VOCAB = 1000000
DIM = 50
DIMP = 56
B = 4096
L = 200

NC = 2
NS = 16
NW = NC * NS

N_TOK = B * L
PER_W = N_TOK // NW
CHUNK = 128
NCH = PER_W // CHUNK


def _body(idx_hbm, table_hbm, out_hbm, idx_v, rows_v, gsem):
    wid = lax.axis_index("s") * NC + lax.axis_index("c")
    base = wid * PER_W
    pltpu.sync_copy(idx_hbm.at[wid], idx_v)

    def step(j, carry):
        pltpu.async_copy(table_hbm.at[idx_v.at[j]], rows_v, gsem).wait()
        pltpu.sync_copy(rows_v, out_hbm.at[pl.ds(base + j * CHUNK, CHUNK)])
        return carry

    lax.fori_loop(0, NCH, step, 0)


@jax.jit
def kernel(indices, table):
    idx = indices.reshape(NW, NCH, CHUNK).astype(jnp.int32)
    tab56 = jnp.pad(table, ((0, 0), (0, DIMP - DIM)))
    mesh = plsc.VectorSubcoreMesh(core_axis_name="c", subcore_axis_name="s")
    out = pl.kernel(
        _body,
        mesh=mesh,
        compiler_params=pltpu.CompilerParams(use_tc_tiling_on_sc=False),
        out_type=jax.ShapeDtypeStruct((N_TOK, DIMP), jnp.float32),
        scratch_types=[
            pltpu.VMEM((NCH, CHUNK), jnp.int32),
            pltpu.VMEM((CHUNK, DIMP), jnp.float32),
            pltpu.SemaphoreType.DMA,
        ],
    )(idx, tab56)
    return out[:, :DIM].reshape(B, L, DIM)


# speedup vs baseline: 1.3722x; 1.3602x over previous
import jax
import jax.numpy as jnp
from jax import lax
from jax.experimental import pallas as pl
from jax.experimental.pallas import tpu as pltpu
from jax.experimental.pallas import tpu_sc as plsc

VOCAB = 1000000
DIM = 50
ROWP = 128
B = 4096
L = 200

NC = 2
NS = 16
NW = NC * NS

N_TOK = B * L
PER_W = N_TOK // NW
CHUNK = 128
NCH = PER_W // CHUNK

VB = 512


def _tc_transpose(tab_t):
    def body(in_ref, out_ref):
        xt = jnp.swapaxes(in_ref[...], 0, 1)
        out_ref[...] = jnp.concatenate(
            [xt, jnp.zeros((VB, ROWP - DIM), jnp.float32)], axis=1)

    grid = (VOCAB + VB - 1) // VB
    return pl.pallas_call(
        body,
        grid=(grid,),
        in_specs=[pl.BlockSpec((DIM, VB), lambda i: (0, i))],
        out_specs=pl.BlockSpec((VB, ROWP), lambda i: (i, 0)),
        out_shape=jax.ShapeDtypeStruct((VOCAB, ROWP), jnp.float32),
    )(tab_t)


def _sc_body(idx_hbm, table_hbm, out_hbm, idx_v, rows_v, gsem):
    wid = lax.axis_index("s") * NC + lax.axis_index("c")
    base = wid * PER_W
    pltpu.sync_copy(idx_hbm.at[wid], idx_v)

    def step(j, carry):
        pltpu.async_copy(table_hbm.at[idx_v.at[j]], rows_v, gsem).wait()
        pltpu.sync_copy(rows_v, out_hbm.at[pl.ds(base + j * CHUNK, CHUNK)])
        return carry

    lax.fori_loop(0, NCH, step, 0)


@jax.jit
def kernel(indices, table):
    idx = indices.reshape(NW, NCH, CHUNK).astype(jnp.int32)
    tab128 = _tc_transpose(jnp.swapaxes(table, 0, 1))
    mesh = plsc.VectorSubcoreMesh(core_axis_name="c", subcore_axis_name="s")
    out = pl.kernel(
        _sc_body,
        mesh=mesh,
        compiler_params=pltpu.CompilerParams(use_tc_tiling_on_sc=True),
        out_type=jax.ShapeDtypeStruct((N_TOK, ROWP), jnp.float32),
        scratch_types=[
            pltpu.VMEM((NCH, CHUNK), jnp.int32),
            pltpu.VMEM((CHUNK, ROWP), jnp.float32),
            pltpu.SemaphoreType.DMA,
        ],
    )(idx, tab128)
    return out[:, :DIM].reshape(B, L, DIM)
